# baseline (speedup 1.0000x reference)
import functools

import jax
import jax.numpy as jnp
from jax import lax
from jax.experimental import pallas as pl
from jax.experimental.pallas import tpu as pltpu
from jax.experimental.pallas import tpu_sc as plsc

_NC = 2
_NS = 16
_NW = _NC * _NS
_CHUNK = 128
_K = 4


def _sc_body(x_hbm, o_hbm, buf, sem_in, sem_out):
    wid = lax.axis_index("s") * _NC + lax.axis_index("c")
    rows_per_w = _CHUNK * _K
    base = wid * rows_per_w

    def in_cp(i):
        return pltpu.make_async_copy(
            x_hbm.at[pl.ds(base + i * _CHUNK, _CHUNK), :], buf.at[i % 2], sem_in)

    def out_cp(i):
        return pltpu.make_async_copy(
            buf.at[i % 2], o_hbm.at[pl.ds(base + i * _CHUNK, _CHUNK), :], sem_out)

    in_cp(0).start()
    in_cp(1).start()
    in_cp(0).wait()
    out_cp(0).start()
    in_cp(1).wait()
    out_cp(1).start()
    out_cp(0).wait()
    in_cp(2).start()
    out_cp(1).wait()
    in_cp(3).start()
    in_cp(2).wait()
    out_cp(2).start()
    in_cp(3).wait()
    out_cp(3).start()
    out_cp(2).wait()
    out_cp(3).wait()


def kernel(x):
    mesh = plsc.VectorSubcoreMesh(
        core_axis_name="c", subcore_axis_name="s", num_cores=_NC)
    run = functools.partial(
        pl.kernel,
        mesh=mesh,
        out_type=jax.ShapeDtypeStruct(x.shape, x.dtype),
        scratch_types=[
            pltpu.VMEM((2, _CHUNK, x.shape[1]), x.dtype),
            pltpu.SemaphoreType.DMA,
            pltpu.SemaphoreType.DMA,
        ],
    )(_sc_body)
    return run(x)


# speedup vs baseline: 1.0456x; 1.0456x over previous
import functools

import jax
import jax.numpy as jnp
from jax import lax
from jax.experimental import pallas as pl
from jax.experimental.pallas import tpu as pltpu
from jax.experimental.pallas import tpu_sc as plsc

_NC = 2
_NS = 16
_NW = _NC * _NS
_CHUNK = 256
_K = 2


def _sc_body(x_hbm, o_hbm, buf, sem_in, sem_out):
    wid = lax.axis_index("s") * _NC + lax.axis_index("c")
    rows_per_w = _CHUNK * _K
    base = wid * rows_per_w

    def in_cp(i):
        return pltpu.make_async_copy(
            x_hbm.at[pl.ds(base + i * _CHUNK, _CHUNK), :], buf, sem_in)

    def out_cp(i):
        return pltpu.make_async_copy(
            buf, o_hbm.at[pl.ds(base + i * _CHUNK, _CHUNK), :], sem_out)

    in_cp(0).start()
    in_cp(0).wait()
    out_cp(0).start()
    out_cp(0).wait()
    in_cp(1).start()
    in_cp(1).wait()
    out_cp(1).start()
    out_cp(1).wait()


def kernel(x):
    mesh = plsc.VectorSubcoreMesh(
        core_axis_name="c", subcore_axis_name="s", num_cores=_NC)
    run = functools.partial(
        pl.kernel,
        mesh=mesh,
        out_type=jax.ShapeDtypeStruct(x.shape, x.dtype),
        scratch_types=[
            pltpu.VMEM((_CHUNK, x.shape[1]), x.dtype),
            pltpu.SemaphoreType.DMA,
            pltpu.SemaphoreType.DMA,
        ],
    )(_sc_body)
    return run(x)


# speedup vs baseline: 2.9386x; 2.8106x over previous
import jax
import jax.numpy as jnp
from jax.experimental import pallas as pl
from jax.experimental.pallas import tpu as pltpu

_K = 2
_B = 2


def _ring_body(x_hbm, o_hbm, buf, sem_in, sem_out):
    rows = x_hbm.shape[0]
    c = rows // _K

    def in_cp(i):
        return pltpu.make_async_copy(
            x_hbm.at[pl.ds(i * c, c), :], buf.at[i % _B], sem_in)

    def out_cp(i):
        return pltpu.make_async_copy(
            buf.at[i % _B], o_hbm.at[pl.ds(i * c, c), :], sem_out)

    for i in range(_B):
        in_cp(i).start()
    for i in range(_K):
        in_cp(i).wait()
        out_cp(i).start()
        j = i + _B
        if j < _K:
            out_cp(i).wait()
            in_cp(j).start()
    for i in range(_K - _B, _K):
        out_cp(i).wait()


def kernel(x):
    rows, cols = x.shape
    return pl.pallas_call(
        _ring_body,
        out_shape=jax.ShapeDtypeStruct(x.shape, x.dtype),
        in_specs=[pl.BlockSpec(memory_space=pl.ANY)],
        out_specs=pl.BlockSpec(memory_space=pl.ANY),
        scratch_shapes=[
            pltpu.VMEM((_B, rows // _K, cols), x.dtype),
            pltpu.SemaphoreType.DMA,
            pltpu.SemaphoreType.DMA,
        ],
    )(x)
